# Initial kernel scaffold; baseline (speedup 1.0000x reference)
import jax
import jax.numpy as jnp
from jax.experimental import pallas as pl


def kernel(x, edge_index, edge_weight, W0, b0, W1, b1, Wk, Wq):
    raise NotImplementedError("write your pallas kernel here")


# speedup vs baseline: 2.9703x; 2.9703x over previous
import functools

import jax
import jax.numpy as jnp
from jax import lax
from jax.experimental import pallas as pl
from jax.experimental.pallas import tpu as pltpu
from jax.experimental.pallas import tpu_sc as plsc

_N = 10000
_E = 320000
_G = 4
_NC = 2
_NS = 16
_LANES = 16
_CHUNK = 128
_NCHUNK = _E // _CHUNK
_ROWS_PER_TILE = _N // _NS
_ZROWS = 632
_GENES_PER_SC = _G // _NC


def _matmul_body(h_ref, w_ref, o_ref):
    o_ref[...] = jnp.dot(h_ref[...], w_ref[0],
                         preferred_element_type=jnp.float32)[None]


def _tc_matmul(h, W, bn=2000):
    g, dk, dout = W.shape
    n = h.shape[0]
    return pl.pallas_call(
        _matmul_body,
        grid=(g, n // bn),
        in_specs=[
            pl.BlockSpec((bn, dk), lambda gi, i: (i, 0)),
            pl.BlockSpec((1, dk, dout), lambda gi, i: (gi, 0, 0)),
        ],
        out_specs=pl.BlockSpec((1, bn, dout), lambda gi, i: (gi, i, 0)),
        out_shape=jax.ShapeDtypeStruct((g, n, dout), jnp.float32),
    )(h, W)


def _spmm_body(d, sup_ref, ei_ref, ew_ref, b_ref, z_ref,
               out_ref, sums_ref,
               acc, rowb, colb, wb, gbuf, obuf, bias, sbuf, gsem):
    cid = lax.axis_index("c")
    sid = lax.axis_index("s")
    nsl = d // _LANES

    base_chunks = _NCHUNK // _NS
    extra = _NCHUNK - base_chunks * _NS
    start = sid * base_chunks + jnp.minimum(sid, extra)
    count = base_chunks + jnp.where(sid < extra, 1, 0)

    rstart = sid * 624 + 8 * jnp.minimum(sid, 2)
    tail_groups = (624 - 512) // 8 + jnp.where(sid < 2, 1, 0)

    for gi in range(_GENES_PER_SC):
        g = cid * _GENES_PER_SC + gi

        zstart = jnp.minimum(rstart, _N - _ZROWS)
        pltpu.sync_copy(z_ref, acc.at[pl.ds(zstart, _ZROWS)])
        pltpu.sync_copy(b_ref.at[pl.ds(g * d, d)], bias)
        plsc.subcore_barrier()

        def chunk_body(c, _):
            eb = c * _CHUNK
            pltpu.sync_copy(ei_ref.at[pl.ds((2 * g) * _E + eb, _CHUNK)], rowb)
            pltpu.sync_copy(ei_ref.at[pl.ds((2 * g + 1) * _E + eb, _CHUNK)],
                            colb)
            pltpu.sync_copy(ew_ref.at[pl.ds(g * _E + eb, _CHUNK)], wb)
            off = g * _N
            for j in range(_CHUNK // _LANES):
                sl = pl.ds(j * _LANES, _LANES)
                colb[sl] = colb[sl] + off
            pltpu.async_copy(sup_ref.at[colb], gbuf, gsem).wait()

            def mul_body(kb, carry):
                wv = wb[pl.ds(kb * _LANES, _LANES)]
                for j in range(_LANES):
                    w = wv[j]
                    k = kb * _LANES + j
                    for m in range(nsl):
                        sl = pl.ds(m * _LANES, _LANES)
                        gbuf[k, sl] = gbuf[k, sl] * w
                return carry

            lax.fori_loop(0, _CHUNK // _LANES, mul_body, 0)
            pltpu.sync_copy(gbuf, acc.at[rowb], add=True)
            return _

        lax.fori_loop(start, start + count, chunk_body, 0)
        plsc.subcore_barrier()

        def flush(roff, nrows, s):
            pltpu.sync_copy(acc.at[pl.ds(rstart + roff, nrows)],
                            obuf.at[pl.ds(0, nrows)])

            def row_body(r, sv):
                for j in range(nsl):
                    sl = pl.ds(j * _LANES, _LANES)
                    v = jnp.maximum(obuf[r, sl] + bias[sl], 0.0)
                    obuf[r, sl] = v
                    sv = sv + v
                return sv

            s = lax.fori_loop(0, nrows, row_body, s)
            pltpu.sync_copy(obuf.at[pl.ds(0, nrows)],
                            out_ref.at[g, pl.ds(rstart + roff, nrows)])
            return s

        s = jnp.zeros((_LANES,), jnp.float32)
        for i in range(4):
            s = flush(i * _CHUNK, _CHUNK, s)

        def tail_body(t, sv):
            return flush(512 + t * 8, 8, sv)

        s = lax.fori_loop(0, tail_groups, tail_body, s)
        sbuf[...] = s
        pltpu.sync_copy(sbuf,
                        sums_ref.at[pl.ds((g * _NS + sid) * _LANES, _LANES)])
        plsc.subcore_barrier()


def _sc_spmm(support, ei_flat, ew_flat, b_flat, d):
    mesh = plsc.VectorSubcoreMesh(core_axis_name="c", subcore_axis_name="s",
                                  num_cores=_NC, num_subcores=_NS)
    zeros = jnp.zeros((_ZROWS, d), jnp.float32)
    run = pl.kernel(
        functools.partial(_spmm_body, d),
        out_type=(
            jax.ShapeDtypeStruct((_G, _N, d), jnp.float32),
            jax.ShapeDtypeStruct((_G * _NS * _LANES,), jnp.float32),
        ),
        mesh=mesh,
        compiler_params=pltpu.CompilerParams(use_tc_tiling_on_sc=False),
        scratch_types=[
            pltpu.VMEM_SHARED((_N, d), jnp.float32),
            pltpu.VMEM((_CHUNK,), jnp.int32),
            pltpu.VMEM((_CHUNK,), jnp.int32),
            pltpu.VMEM((_CHUNK,), jnp.float32),
            pltpu.VMEM((_CHUNK, d), jnp.float32),
            pltpu.VMEM((_CHUNK, d), jnp.float32),
            pltpu.VMEM((d,), jnp.float32),
            pltpu.VMEM((_LANES,), jnp.float32),
            pltpu.SemaphoreType.DMA,
        ],
    )
    return run(support, ei_flat, ew_flat, b_flat, zeros)


def _combine_body(d, sums_ref, wk_ref, wq_ref, h_ref, o_ref):
    s = jnp.sum(sums_ref[...], axis=1) * (1.0 / (_N * d))
    k = jnp.maximum(jnp.dot(s.reshape(1, _G), wk_ref[...],
                            preferred_element_type=jnp.float32), 0.0)
    logits = jnp.dot(k, wq_ref[...], preferred_element_type=jnp.float32)
    att = jax.nn.softmax(logits, axis=1)
    o_ref[...] = jnp.sum(h_ref[...] * att.reshape(_G, 1, 1), axis=0)


def _tc_combine(h_relu, sums, wk, wq, bn=2000):
    g, n, d = h_relu.shape
    return pl.pallas_call(
        functools.partial(_combine_body, d),
        grid=(n // bn,),
        in_specs=[
            pl.BlockSpec((g, _NS * _LANES), lambda i: (0, 0)),
            pl.BlockSpec((g, g), lambda i: (0, 0)),
            pl.BlockSpec((g, g), lambda i: (0, 0)),
            pl.BlockSpec((g, bn, d), lambda i: (0, i, 0)),
        ],
        out_specs=pl.BlockSpec((bn, d), lambda i: (i, 0)),
        out_shape=jax.ShapeDtypeStruct((n, d), jnp.float32),
    )(sums, wk, wq, h_relu)


def kernel(x, edge_index, edge_weight, W0, b0, W1, b1, Wk, Wq):
    ei_flat = edge_index.reshape(_G * 2 * _E)
    ew_flat = edge_weight.reshape(_G * _E)
    h = x
    for l, (W, b) in enumerate(((W0, b0), (W1, b1))):
        d = W.shape[2]
        support = _tc_matmul(h, W)
        h_relu, sums = _sc_spmm(support.reshape(_G * _N, d),
                                ei_flat, ew_flat, b.reshape(_G * d), d)
        h = _tc_combine(h_relu, sums.reshape(_G, _NS * _LANES),
                        Wk[l], Wq[l])
    return h
